# baseline (speedup 1.0000x reference)
import functools

import jax
import jax.numpy as jnp
from jax import lax
from jax.experimental import pallas as pl
from jax.experimental.pallas import tpu as pltpu
from jax.experimental.pallas import tpu_sc as plsc

N = 10000
E = 320000
D = 128

NC = 2
NS = 16
CH = 128
CHUNKS = 2560
E_PAD = CHUNKS * CH
RPS = 632
NPAD = NS * RPS

_BM = 1000


def _mm3_body(x_ref, w_ref, b_ref, h0_ref, h1_ref, h2_ref):
    h = jnp.dot(x_ref[...], w_ref[...],
                preferred_element_type=jnp.float32,
                precision=lax.Precision.HIGHEST) + b_ref[...]
    h0_ref[...] = h[:, 0:D]
    h1_ref[...] = h[:, D:2 * D]
    h2_ref[...] = h[:, 2 * D:3 * D]


def _mm3(x, w, b):
    return pl.pallas_call(
        _mm3_body,
        grid=(N // _BM,),
        in_specs=[
            pl.BlockSpec((_BM, D), lambda i: (i, 0)),
            pl.BlockSpec((D, 3 * D), lambda i: (0, 0)),
            pl.BlockSpec((1, 3 * D), lambda i: (0, 0)),
        ],
        out_specs=[pl.BlockSpec((_BM, D), lambda i: (i, 0))] * 3,
        out_shape=[jax.ShapeDtypeStruct((N, D), jnp.float32)] * 3,
    )(x, w, b)


def _assemble_body(h0_ref, y1_ref, p0_ref, p1_ref, out_ref):
    out_ref[:, 0:D] = h0_ref[...]
    out_ref[:, D:2 * D] = y1_ref[...]
    out_ref[:, 2 * D:3 * D] = p0_ref[...] + p1_ref[...]


def _assemble(h0, y1, p0, p1):
    return pl.pallas_call(
        _assemble_body,
        grid=(N // _BM,),
        in_specs=[pl.BlockSpec((_BM, D), lambda i: (i, 0))] * 4,
        out_specs=pl.BlockSpec((_BM, 3 * D), lambda i: (i, 0)),
        out_shape=jax.ShapeDtypeStruct((N, 3 * D), jnp.float32),
    )(h0, y1, p0, p1)


def _pad_idx_body(ei_ref, out_ref):
    out_ref[:, :E] = ei_ref[...]
    i = lax.broadcasted_iota(jnp.int32, (1, E_PAD - E), 1)
    out_ref[0:1, E:] = N + lax.rem(i, NPAD - N)
    out_ref[1:2, E:] = jnp.zeros((1, E_PAD - E), jnp.int32)


def _pad_idx(ei):
    return pl.pallas_call(
        _pad_idx_body,
        out_shape=jax.ShapeDtypeStruct((2, E_PAD), jnp.int32),
    )(ei)


_mesh = plsc.VectorSubcoreMesh(core_axis_name="c", subcore_axis_name="s",
                               num_cores=NC, num_subcores=NS)

_SC_SCRATCH = [
    pltpu.VMEM((CH,), jnp.int32),
    pltpu.VMEM((CH,), jnp.int32),
    pltpu.VMEM((CH, D), jnp.float32),
    pltpu.VMEM_SHARED((NPAD, D), jnp.float32),
    pltpu.SemaphoreType.DMA,
    pltpu.SemaphoreType.DMA,
    pltpu.SemaphoreType.DMA,
]

_SC_OUT2 = (jax.ShapeDtypeStruct((NPAD, D), jnp.float32),
            jax.ShapeDtypeStruct((NPAD, D), jnp.float32))


def _zero_acc(rowsv, acc, s):
    @pl.loop(0, CH)
    def _(r):
        @pl.loop(0, D, step=16)
        def _(k):
            rowsv[r, pl.ds(k, 16)] = jnp.zeros((16,), jnp.float32)

    @pl.loop(0, 4)
    def _(j):
        pltpu.sync_copy(rowsv, acc.at[pl.ds(s * RPS + j * CH, CH)])

    pltpu.sync_copy(rowsv.at[pl.ds(0, 120)],
                    acc.at[pl.ds(s * RPS + 4 * CH, 120)])


def _edge_loop(h_hbm, row_hbm, col_hbm, colv, rowv, rowsv, acc,
               semc, semr, semg, start, step):
    @pl.loop(start, CHUNKS, step=step)
    def _(i):
        base = i * CH
        dc = pltpu.async_copy(col_hbm.at[pl.ds(base, CH)], colv, semc)
        dr = pltpu.async_copy(row_hbm.at[pl.ds(base, CH)], rowv, semr)
        dc.wait()
        dg = pltpu.async_copy(h_hbm.at[colv], rowsv, semg)
        dr.wait()
        dg.wait()
        pltpu.sync_copy(rowsv, acc.at[rowv], add=True)
    return None


@functools.partial(pl.kernel, out_type=_SC_OUT2, mesh=_mesh,
                   scratch_types=_SC_SCRATCH)
def _spmm_pass1(h1_hbm, h2_hbm, row_hbm, col_hbm, y1_hbm, y2_hbm,
                colv, rowv, rowsv, acc, semc, semr, semg):
    c = lax.axis_index("c")
    s = lax.axis_index("s")
    _zero_acc(rowsv, acc, s)
    plsc.subcore_barrier()

    @pl.when(c == 0)
    def _():
        _edge_loop(h1_hbm, row_hbm, col_hbm, colv, rowv, rowsv, acc, semc, semr, semg,
                   s, NS)

    @pl.when(c == 1)
    def _():
        _edge_loop(h2_hbm, row_hbm, col_hbm, colv, rowv, rowsv, acc, semc, semr, semg,
                   s, NS)

    plsc.subcore_barrier()

    @pl.when(c == 0)
    def _():
        pltpu.sync_copy(acc.at[pl.ds(s * RPS, RPS)],
                        y1_hbm.at[pl.ds(s * RPS, RPS)])

    @pl.when(c == 1)
    def _():
        pltpu.sync_copy(acc.at[pl.ds(s * RPS, RPS)],
                        y2_hbm.at[pl.ds(s * RPS, RPS)])


@functools.partial(pl.kernel, out_type=_SC_OUT2, mesh=_mesh,
                   scratch_types=_SC_SCRATCH)
def _spmm_pass2(h_hbm, row_hbm, col_hbm, p0_hbm, p1_hbm,
                colv, rowv, rowsv, acc, semc, semr, semg):
    c = lax.axis_index("c")
    s = lax.axis_index("s")
    _zero_acc(rowsv, acc, s)
    plsc.subcore_barrier()
    _edge_loop(h_hbm, row_hbm, col_hbm, colv, rowv, rowsv, acc, semc, semr, semg,
               s * NC + c, NC * NS)
    plsc.subcore_barrier()

    @pl.when(c == 0)
    def _():
        pltpu.sync_copy(acc.at[pl.ds(s * RPS, RPS)],
                        p0_hbm.at[pl.ds(s * RPS, RPS)])

    @pl.when(c == 1)
    def _():
        pltpu.sync_copy(acc.at[pl.ds(s * RPS, RPS)],
                        p1_hbm.at[pl.ds(s * RPS, RPS)])


def kernel(x, edge_index, W0, b0, W1, b1, W2, b2):
    eip = _pad_idx(edge_index.astype(jnp.int32))
    row, col = eip[0], eip[1]
    w = jnp.concatenate([W0.T, W1.T, W2.T], axis=1)
    b = jnp.concatenate([b0, b1, b2]).reshape(1, 3 * D)
    h0, h1, h2 = _mm3(x, w, b)
    y1, y2a = _spmm_pass1(h1, h2, row, col)
    p0, p1 = _spmm_pass2(y2a, row, col)
    return _assemble(h0, y1, p0, p1)


# speedup vs baseline: 1.9614x; 1.9614x over previous
import functools

import jax
import jax.numpy as jnp
from jax import lax
from jax.experimental import pallas as pl
from jax.experimental.pallas import tpu as pltpu
from jax.experimental.pallas import tpu_sc as plsc

N = 10000
E = 320000
D = 128

NC = 2
NS = 16
CH = 128
CHUNKS = E // CH
RPS = 632
NPAD = NS * RPS

_BM = 1000


def _mm3_body(x_ref, w_ref, b_ref, h0_ref, h1_ref, h2_ref):
    h = jnp.dot(x_ref[...], w_ref[...],
                preferred_element_type=jnp.float32,
                precision=lax.Precision.HIGHEST) + b_ref[...]
    h0_ref[...] = h[:, 0:D]
    h1_ref[...] = h[:, D:2 * D]
    h2_ref[...] = h[:, 2 * D:3 * D]


def _mm3(x, w, b):
    return pl.pallas_call(
        _mm3_body,
        grid=(N // _BM,),
        in_specs=[
            pl.BlockSpec((_BM, D), lambda i: (i, 0)),
            pl.BlockSpec((D, 3 * D), lambda i: (0, 0)),
            pl.BlockSpec((1, 3 * D), lambda i: (0, 0)),
        ],
        out_specs=[pl.BlockSpec((_BM, D), lambda i: (i, 0))] * 3,
        out_shape=[jax.ShapeDtypeStruct((N, D), jnp.float32)] * 3,
    )(x, w, b)


def _assemble_body(h0_ref, y1_ref, p0_ref, p1_ref, out_ref):
    out_ref[:, 0:D] = h0_ref[...]
    out_ref[:, D:2 * D] = y1_ref[...]
    out_ref[:, 2 * D:3 * D] = p0_ref[...] + p1_ref[...]


def _assemble(h0, y1, p0, p1):
    return pl.pallas_call(
        _assemble_body,
        grid=(N // _BM,),
        in_specs=[pl.BlockSpec((_BM, D), lambda i: (i, 0))] * 4,
        out_specs=pl.BlockSpec((_BM, 3 * D), lambda i: (i, 0)),
        out_shape=jax.ShapeDtypeStruct((N, 3 * D), jnp.float32),
    )(h0, y1, p0, p1)


_mesh = plsc.VectorSubcoreMesh(core_axis_name="c", subcore_axis_name="s",
                               num_cores=NC, num_subcores=NS)

_SC_SCRATCH = [
    pltpu.VMEM((CH,), jnp.int32),
    pltpu.VMEM((CH,), jnp.int32),
    pltpu.VMEM((CH,), jnp.int32),
    pltpu.VMEM((CH,), jnp.int32),
    pltpu.VMEM((CH, D), jnp.float32),
    pltpu.VMEM_SHARED((NPAD, D), jnp.float32),
    pltpu.SemaphoreType.DMA,
    pltpu.SemaphoreType.DMA,
    pltpu.SemaphoreType.DMA,
    pltpu.SemaphoreType.DMA,
    pltpu.SemaphoreType.DMA,
]

_SC_OUT2 = (jax.ShapeDtypeStruct((NPAD, D), jnp.float32),
            jax.ShapeDtypeStruct((NPAD, D), jnp.float32))


def _zero_acc(rowsv, acc, s):
    @pl.loop(0, CH)
    def _(r):
        @pl.loop(0, D, step=16)
        def _(k):
            rowsv[r, pl.ds(k, 16)] = jnp.zeros((16,), jnp.float32)

    @pl.loop(0, 4)
    def _(j):
        pltpu.sync_copy(rowsv, acc.at[pl.ds(s * RPS + j * CH, CH)])

    pltpu.sync_copy(rowsv.at[pl.ds(0, 120)],
                    acc.at[pl.ds(s * RPS + 4 * CH, 120)])


def _edge_loop(h_hbm, row_hbm, col_hbm, colv, rowv, colv1, rowv1, rowsv,
               acc, semc, semr, semc1, semr1, semg, start, step):
    count = (CHUNKS - start + step - 1) // step
    npairs = count // 2

    @pl.loop(0, npairs)
    def _(t):
        b0 = (start + (2 * t) * step) * CH
        b1 = b0 + step * CH
        dc0 = pltpu.async_copy(col_hbm.at[pl.ds(b0, CH)], colv, semc)
        dr0 = pltpu.async_copy(row_hbm.at[pl.ds(b0, CH)], rowv, semr)
        dc1 = pltpu.async_copy(col_hbm.at[pl.ds(b1, CH)], colv1, semc1)
        dr1 = pltpu.async_copy(row_hbm.at[pl.ds(b1, CH)], rowv1, semr1)
        dc0.wait()
        dg0 = pltpu.async_copy(h_hbm.at[colv], rowsv, semg)
        dr0.wait()
        dg0.wait()
        pltpu.sync_copy(rowsv, acc.at[rowv], add=True)
        dc1.wait()
        dg1 = pltpu.async_copy(h_hbm.at[colv1], rowsv, semg)
        dr1.wait()
        dg1.wait()
        pltpu.sync_copy(rowsv, acc.at[rowv1], add=True)

    @pl.when(count % 2 == 1)
    def _():
        base = (start + (count - 1) * step) * CH
        dc = pltpu.async_copy(col_hbm.at[pl.ds(base, CH)], colv, semc)
        dr = pltpu.async_copy(row_hbm.at[pl.ds(base, CH)], rowv, semr)
        dc.wait()
        dg = pltpu.async_copy(h_hbm.at[colv], rowsv, semg)
        dr.wait()
        dg.wait()
        pltpu.sync_copy(rowsv, acc.at[rowv], add=True)
    return None


@functools.partial(pl.kernel, out_type=_SC_OUT2, mesh=_mesh,
                   scratch_types=_SC_SCRATCH)
def _spmm_pass1(h1_hbm, h2_hbm, row_hbm, col_hbm, y1_hbm, y2_hbm,
                colv, rowv, colv1, rowv1, rowsv, acc,
                semc, semr, semc1, semr1, semg):
    c = lax.axis_index("c")
    s = lax.axis_index("s")
    _zero_acc(rowsv, acc, s)
    plsc.subcore_barrier()

    @pl.when(c == 0)
    def _():
        _edge_loop(h1_hbm, row_hbm, col_hbm, colv, rowv, colv1, rowv1,
                   rowsv, acc, semc, semr, semc1, semr1, semg, s, NS)

    @pl.when(c == 1)
    def _():
        _edge_loop(h2_hbm, row_hbm, col_hbm, colv, rowv, colv1, rowv1,
                   rowsv, acc, semc, semr, semc1, semr1, semg, s, NS)

    plsc.subcore_barrier()

    @pl.when(c == 0)
    def _():
        pltpu.sync_copy(acc.at[pl.ds(s * RPS, RPS)],
                        y1_hbm.at[pl.ds(s * RPS, RPS)])

    @pl.when(c == 1)
    def _():
        pltpu.sync_copy(acc.at[pl.ds(s * RPS, RPS)],
                        y2_hbm.at[pl.ds(s * RPS, RPS)])


@functools.partial(pl.kernel, out_type=_SC_OUT2, mesh=_mesh,
                   scratch_types=_SC_SCRATCH)
def _spmm_pass2(h_hbm, row_hbm, col_hbm, p0_hbm, p1_hbm,
                colv, rowv, colv1, rowv1, rowsv, acc,
                semc, semr, semc1, semr1, semg):
    c = lax.axis_index("c")
    s = lax.axis_index("s")
    _zero_acc(rowsv, acc, s)
    plsc.subcore_barrier()
    _edge_loop(h_hbm, row_hbm, col_hbm, colv, rowv, colv1, rowv1,
               rowsv, acc, semc, semr, semc1, semr1, semg,
               s * NC + c, NC * NS)
    plsc.subcore_barrier()

    @pl.when(c == 0)
    def _():
        pltpu.sync_copy(acc.at[pl.ds(s * RPS, RPS)],
                        p0_hbm.at[pl.ds(s * RPS, RPS)])

    @pl.when(c == 1)
    def _():
        pltpu.sync_copy(acc.at[pl.ds(s * RPS, RPS)],
                        p1_hbm.at[pl.ds(s * RPS, RPS)])


def kernel(x, edge_index, W0, b0, W1, b1, W2, b2):
    ei = edge_index.astype(jnp.int32)
    row, col = ei[0], ei[1]
    w = jnp.concatenate([W0.T, W1.T, W2.T], axis=1)
    b = jnp.concatenate([b0, b1, b2]).reshape(1, 3 * D)
    h0, h1, h2 = _mm3(x, w, b)
    y1, y2a = _spmm_pass1(h1, h2, row, col)
    p0, p1 = _spmm_pass2(y2a, row, col)
    return _assemble(h0, y1, p0, p1)
